# Initial kernel scaffold; baseline (speedup 1.0000x reference)
import jax
import jax.numpy as jnp
from jax.experimental import pallas as pl


def kernel(images, g_task, W1, b1, W2, b2, ln1_g, ln1_b, Wgate, Wgp, A_log, Dp, WB, bB, WC, bC, Wd, bd, Wo, bo, lno_g, lno_b):
    raise NotImplementedError("write your pallas kernel here")


# speedup vs baseline: 1092.3241x; 1092.3241x over previous
import functools
import numpy as np
import jax
import jax.numpy as jnp
from jax import lax
from jax.experimental import pallas as pl
from jax.experimental.pallas import tpu as pltpu

PS = 16
D_F = 256
D_STATE = 16
ALPHA = 0.5
GH = 14
GW = 14
P = GH * GW
NB = 2
ROWS = NB * P
E = 2 * GH * GW - GH - GW
EPAD = 384
PPAD = 224
NCHUNKS_E = EPAD // 16
NCHUNKS_P = PPAD // 16


def _grid_edge_list():
    s, d = [], []
    for i in range(GH):
        for j in range(GW):
            u = i * GW + j
            if j + 1 < GW:
                s.append(u)
                d.append(u + 1)
            if i + 1 < GH:
                s.append(u)
                d.append(u + GW)
    return np.array(s, np.int32), np.array(d, np.int32)


_SRC_E, _DST_E = _grid_edge_list()
_IS_R = (_DST_E == _SRC_E + 1)


def _tc_front(x_ref, g_ref, W1_ref, b1_ref, W2_ref, b2_ref, g1_ref, bl1_ref,
              Wgate_ref, Wgp_ref, Wd_ref, bd_ref, Wbc_ref, bbc_ref,
              x_out, delta_out, bc_out, sem_out, wr_out, wd_out):
    x = x_ref[...]
    h = jnp.dot(x, W1_ref[...], preferred_element_type=jnp.float32) + b1_ref[...]
    h = 0.5 * h * (1.0 + lax.erf(h * np.float32(1.0 / np.sqrt(2.0))))
    f = jnp.dot(h, W2_ref[...], preferred_element_type=jnp.float32) + b2_ref[...]
    m = f.mean(-1, keepdims=True)
    v = ((f - m) ** 2).mean(-1, keepdims=True)
    f = (f - m) * lax.rsqrt(v + np.float32(1e-5)) * g1_ref[...] + bl1_ref[...]

    g = g_ref[...]
    gg = jnp.dot(g, Wgate_ref[...], preferred_element_type=jnp.float32)
    gp = jnp.dot(g, Wgp_ref[...], preferred_element_type=jnp.float32)
    gn = gg / jnp.maximum(jnp.sqrt((gg * gg).sum(-1, keepdims=True)), np.float32(1e-12))
    fn = f / jnp.maximum(jnp.sqrt((f * f).sum(-1, keepdims=True)), np.float32(1e-12))

    ridx = lax.broadcasted_iota(jnp.int32, (ROWS, D_F), 0)
    img0 = ridx < P
    gnsel = jnp.where(img0, gn[0:1, :], gn[1:2, :])
    gpsel = jnp.where(img0, gp[0:1, :], gp[1:2, :])
    sem = (fn * gnsel).sum(-1, keepdims=True)
    X = f + sem * gpsel

    xw = jnp.dot(X, Wd_ref[...], preferred_element_type=jnp.float32) + bd_ref[...]
    delta = jnp.maximum(xw, 0.0) + jnp.log1p(jnp.exp(-jnp.abs(xw)))
    bc = jnp.dot(X, Wbc_ref[...], preferred_element_type=jnp.float32) + bbc_ref[...]

    wr = (fn[:-1] * fn[1:]).sum(-1, keepdims=True) + \
        np.float32(ALPHA) * (sem[:-1] * sem[1:])
    wd = (fn[:-GW] * fn[GW:]).sum(-1, keepdims=True) + \
        np.float32(ALPHA) * (sem[:-GW] * sem[GW:])

    x_out[...] = X
    delta_out[...] = delta
    bc_out[...] = bc
    sem_out[...] = sem
    wr_out[...] = jnp.concatenate([wr, jnp.zeros((1, 1), jnp.float32)], axis=0)
    wd_out[...] = jnp.concatenate([wd, jnp.zeros((GW, 1), jnp.float32)], axis=0)


def _tc_out(y_ref, Wo_ref, bo_ref, g_ref, b_ref, o_ref):
    o = jnp.dot(y_ref[...], Wo_ref[...], preferred_element_type=jnp.float32) + bo_ref[...]
    m = o.mean(-1, keepdims=True)
    v = ((o - m) ** 2).mean(-1, keepdims=True)
    o_ref[...] = (o - m) * lax.rsqrt(v + np.float32(1e-5)) * g_ref[...] + b_ref[...]


def _tc_tree(w_v_ref, w_s_ref, src_ref, dst_ref, sem_ref, delta_ref, x_ref,
             bc_ref, at_ref, dp_ref, y_ref,
             su_s, dv_s, par_s, deg_s, adj_s, vis_s, ord_s, pb_s, hs_v):
    iota_e = lax.broadcasted_iota(jnp.int32, (1, EPAD), 1)
    iota_p = lax.broadcasted_iota(jnp.int32, (1, 256), 1)
    dp_row = dp_ref[...]
    aneg = [-jnp.exp(at_ref[pl.ds(st, 1), :]) for st in range(D_STATE)]

    for b in range(NB):
        boff = b * P
        wrow = w_v_ref[pl.ds(b, 1), :]

        def _rank_body(f, pos):
            wf = w_s_ref[b, f]
            prec = (wf > wrow) | ((wf == wrow) & (f < iota_e))
            return pos + jnp.where(prec, 1, 0).astype(jnp.int32)

        pos = lax.fori_loop(0, E, _rank_body,
                            jnp.zeros((1, EPAD), jnp.int32))

        def _inv_body(r, _):
            er = jnp.sum(jnp.where(pos == r, iota_e, 0)).astype(jnp.int32)
            su_s[r] = src_ref[er]
            dv_s[r] = dst_ref[er]
            return 0

        lax.fori_loop(0, E, _inv_body, 0)

        def _init_body(i, _):
            par_s[i] = i
            deg_s[i] = 0
            vis_s[i] = 0
            adj_s[4 * i] = 0
            adj_s[4 * i + 1] = 0
            adj_s[4 * i + 2] = 0
            adj_s[4 * i + 3] = 0
            return 0

        lax.fori_loop(0, P, _init_body, 0)

        def _find(x0):
            def cond(carry):
                xx, pp = carry
                return pp != xx

            def body(carry):
                xx, pp = carry
                g = par_s[pp]
                par_s[xx] = g
                return g, par_s[g]

            root, _ = lax.while_loop(cond, body, (x0, par_s[x0]))
            return root

        def _kruskal_body(e, _):
            u = su_s[e]
            v = dv_s[e]
            ru = _find(u)
            rv = _find(v)

            @pl.when(ru != rv)
            def _():
                par_s[ru] = rv
                du = deg_s[u]
                adj_s[u * 4 + du] = v
                deg_s[u] = du + 1
                dvn = deg_s[v]
                adj_s[v * 4 + dvn] = u
                deg_s[v] = dvn + 1

            return 0

        lax.fori_loop(0, E, _kruskal_body, 0)

        semrow = sem_ref[pl.ds(b, 1), :]
        mx = jnp.max(semrow)
        root = jnp.min(jnp.where(semrow == mx, iota_p, 100000)).astype(jnp.int32)

        ord_s[0] = root
        pb_s[0] = jnp.int32(-1)
        vis_s[root] = jnp.int32(1)

        def _bfs_body(i, tail):
            v = ord_s[i]
            dvn = deg_s[v]
            for k in range(4):
                u = adj_s[v * 4 + k]
                take = (k < dvn) & (vis_s[u] == 0)

                @pl.when(take)
                def _():
                    vis_s[u] = jnp.int32(1)
                    ord_s[tail] = u
                    pb_s[tail] = i

                tail = jnp.where(take, tail + 1, tail)
            return tail

        lax.fori_loop(0, P, _bfs_body, jnp.int32(1))

        def _scan_body(i, _):
            v = ord_s[i]
            pbi = pb_s[i]
            row = boff + v
            dvec = delta_ref[pl.ds(row, 1), :]
            xvec = x_ref[pl.ds(row, 1), :]
            bcrow = bc_ref[pl.ds(row, 1), :]
            dx = dvec * xvec
            rooted = pbi >= 0
            hpb = jnp.maximum(pbi, 0) * D_STATE
            hib = i * D_STATE
            y = dp_row * xvec
            for st in range(D_STATE):
                ab = jnp.exp(dvec * aneg[st])
                hp = jnp.where(rooted, hs_v[pl.ds(hpb + st, 1), :],
                               jnp.float32(0.0))
                hcur = ab * hp + dx * bcrow[0, st]
                hs_v[pl.ds(hib + st, 1), :] = hcur
                y = y + hcur * bcrow[0, st + D_STATE]
            y_ref[pl.ds(row, 1), :] = y
            return 0

        lax.fori_loop(0, P, _scan_body, 0)


def _run_tc_tree(w_edges, sem2, delta, X, bc, at2, dp_row):
    src_c = jnp.pad(jnp.asarray(_SRC_E), (0, EPAD - E))
    dst_c = jnp.pad(jnp.asarray(_DST_E), (0, EPAD - E))
    smem = pl.BlockSpec(memory_space=pltpu.SMEM)
    vm = pl.BlockSpec()
    return pl.pallas_call(
        _tc_tree,
        out_shape=jax.ShapeDtypeStruct((ROWS, D_F), jnp.float32),
        in_specs=[vm, smem, smem, smem, vm, vm, vm, vm, vm, vm],
        scratch_shapes=[
            pltpu.SMEM((EPAD,), jnp.int32),
            pltpu.SMEM((EPAD,), jnp.int32),
            pltpu.SMEM((PPAD,), jnp.int32),
            pltpu.SMEM((PPAD,), jnp.int32),
            pltpu.SMEM((4 * PPAD,), jnp.int32),
            pltpu.SMEM((PPAD,), jnp.int32),
            pltpu.SMEM((PPAD,), jnp.int32),
            pltpu.SMEM((PPAD,), jnp.int32),
            pltpu.VMEM((P * D_STATE, D_F), jnp.float32),
        ],
    )(w_edges, w_edges, src_c, dst_c, sem2, delta, X, bc, at2, dp_row)


def kernel(images, g_task, W1, b1, W2, b2, ln1_g, ln1_b, Wgate, Wgp, A_log,
           Dp, WB, bB, WC, bC, Wd, bd, Wo, bo, lno_g, lno_b):
    B, C, H, W = images.shape
    nH, nW = H // PS, W // PS
    x = images.reshape(B, C, nH, PS, nW, PS).transpose(0, 2, 4, 1, 3, 5)
    x = x.reshape(B * nH * nW, C * PS * PS)

    Wbc = jnp.concatenate([WB, WC], axis=1)
    bbc = jnp.concatenate([bB, bC]).reshape(1, 32)

    fr = pl.pallas_call(
        _tc_front,
        out_shape=[
            jax.ShapeDtypeStruct((ROWS, D_F), jnp.float32),
            jax.ShapeDtypeStruct((ROWS, D_F), jnp.float32),
            jax.ShapeDtypeStruct((ROWS, 32), jnp.float32),
            jax.ShapeDtypeStruct((ROWS, 1), jnp.float32),
            jax.ShapeDtypeStruct((ROWS, 1), jnp.float32),
            jax.ShapeDtypeStruct((ROWS, 1), jnp.float32),
        ],
    )(x, g_task, W1, b1.reshape(1, -1), W2, b2.reshape(1, -1),
      ln1_g.reshape(1, -1), ln1_b.reshape(1, -1), Wgate, Wgp, Wd,
      bd.reshape(1, -1), Wbc, bbc)
    X, delta, bc, sem, wr, wd = fr

    base = jnp.asarray(np.arange(NB)[:, None] * P + _SRC_E[None, :])
    wre = wr[:, 0][base]
    wde = wd[:, 0][base]
    w_edges = jnp.where(jnp.asarray(_IS_R)[None, :], wre, wde)
    w_edges = jnp.pad(w_edges, ((0, 0), (0, EPAD - E)),
                      constant_values=-1e30)
    sem2 = jnp.pad(sem[:, 0].reshape(NB, P), ((0, 0), (0, 256 - P)),
                   constant_values=-1e30)

    Y = _run_tc_tree(w_edges, sem2, delta, X, bc, A_log.T,
                     Dp.reshape(1, D_F))

    Yr = Y
    out = pl.pallas_call(
        _tc_out,
        out_shape=jax.ShapeDtypeStruct((ROWS, D_F), jnp.float32),
    )(Yr, Wo, bo.reshape(1, -1),
      lno_g.reshape(1, -1), lno_b.reshape(1, -1))
    return out.reshape(NB, P, D_F)


# speedup vs baseline: 1225.1884x; 1.1216x over previous
import functools
import numpy as np
import jax
import jax.numpy as jnp
from jax import lax
from jax.experimental import pallas as pl
from jax.experimental.pallas import tpu as pltpu

PS = 16
D_F = 256
D_STATE = 16
ALPHA = 0.5
GH = 14
GW = 14
P = GH * GW
NB = 2
ROWS = NB * P
E = 2 * GH * GW - GH - GW
EPAD = 384
PPAD = 224
NCHUNKS_E = EPAD // 16
NCHUNKS_P = PPAD // 16


def _grid_edge_list():
    s, d = [], []
    for i in range(GH):
        for j in range(GW):
            u = i * GW + j
            if j + 1 < GW:
                s.append(u)
                d.append(u + 1)
            if i + 1 < GH:
                s.append(u)
                d.append(u + GW)
    return np.array(s, np.int32), np.array(d, np.int32)


_SRC_E, _DST_E = _grid_edge_list()
_IS_R = (_DST_E == _SRC_E + 1)


def _tc_front(x_ref, g_ref, W1_ref, b1_ref, W2_ref, b2_ref, g1_ref, bl1_ref,
              Wgate_ref, Wgp_ref, Wd_ref, bd_ref, Wbc_ref, bbc_ref,
              x_out, delta_out, bc_out, sem_out, wr_out, wd_out):
    x = x_ref[...]
    h = jnp.dot(x, W1_ref[...], preferred_element_type=jnp.float32) + b1_ref[...]
    h = 0.5 * h * (1.0 + lax.erf(h * np.float32(1.0 / np.sqrt(2.0))))
    f = jnp.dot(h, W2_ref[...], preferred_element_type=jnp.float32) + b2_ref[...]
    m = f.mean(-1, keepdims=True)
    v = ((f - m) ** 2).mean(-1, keepdims=True)
    f = (f - m) * lax.rsqrt(v + np.float32(1e-5)) * g1_ref[...] + bl1_ref[...]

    g = g_ref[...]
    gg = jnp.dot(g, Wgate_ref[...], preferred_element_type=jnp.float32)
    gp = jnp.dot(g, Wgp_ref[...], preferred_element_type=jnp.float32)
    gn = gg / jnp.maximum(jnp.sqrt((gg * gg).sum(-1, keepdims=True)), np.float32(1e-12))
    fn = f / jnp.maximum(jnp.sqrt((f * f).sum(-1, keepdims=True)), np.float32(1e-12))

    ridx = lax.broadcasted_iota(jnp.int32, (ROWS, D_F), 0)
    img0 = ridx < P
    gnsel = jnp.where(img0, gn[0:1, :], gn[1:2, :])
    gpsel = jnp.where(img0, gp[0:1, :], gp[1:2, :])
    sem = (fn * gnsel).sum(-1, keepdims=True)
    X = f + sem * gpsel

    xw = jnp.dot(X, Wd_ref[...], preferred_element_type=jnp.float32) + bd_ref[...]
    delta = jnp.maximum(xw, 0.0) + jnp.log1p(jnp.exp(-jnp.abs(xw)))
    bc = jnp.dot(X, Wbc_ref[...], preferred_element_type=jnp.float32) + bbc_ref[...]

    wr = (fn[:-1] * fn[1:]).sum(-1, keepdims=True) + \
        np.float32(ALPHA) * (sem[:-1] * sem[1:])
    wd = (fn[:-GW] * fn[GW:]).sum(-1, keepdims=True) + \
        np.float32(ALPHA) * (sem[:-GW] * sem[GW:])

    x_out[...] = X
    delta_out[...] = delta
    bc_out[...] = bc
    sem_out[...] = sem
    wr_out[...] = jnp.concatenate([wr, jnp.zeros((1, 1), jnp.float32)], axis=0)
    wd_out[...] = jnp.concatenate([wd, jnp.zeros((GW, 1), jnp.float32)], axis=0)


def _tc_out(y_ref, Wo_ref, bo_ref, g_ref, b_ref, o_ref):
    o = jnp.dot(y_ref[...], Wo_ref[...], preferred_element_type=jnp.float32) + bo_ref[...]
    m = o.mean(-1, keepdims=True)
    v = ((o - m) ** 2).mean(-1, keepdims=True)
    o_ref[...] = (o - m) * lax.rsqrt(v + np.float32(1e-5)) * g_ref[...] + b_ref[...]


def _tc_tree(w_v_ref, w_s_ref, src_ref, dst_ref, sem_ref, delta_ref, x_ref,
             bt_ref, ct_ref, at_ref, dp_ref, y_ref,
             su_s, dv_s, par_s, deg_s, adj_s, vis_s, ord_s, pb_s, hs_v):
    iota_e = lax.broadcasted_iota(jnp.int32, (1, EPAD), 1)
    iota_p = lax.broadcasted_iota(jnp.int32, (1, 256), 1)
    dp_row = dp_ref[...]
    aneg_blk = -jnp.exp(at_ref[...])

    for b in range(NB):
        boff = b * P
        wrow = w_v_ref[pl.ds(b, 1), :]

        def _rank_body(f, pos):
            wf = w_s_ref[b, f]
            prec = (wf > wrow) | ((wf == wrow) & (f < iota_e))
            return pos + jnp.where(prec, 1, 0).astype(jnp.int32)

        pos = lax.fori_loop(0, E, _rank_body,
                            jnp.zeros((1, EPAD), jnp.int32))

        def _inv_body(r, _):
            er = jnp.sum(jnp.where(pos == r, iota_e, 0)).astype(jnp.int32)
            su_s[r] = src_ref[er]
            dv_s[r] = dst_ref[er]
            return 0

        lax.fori_loop(0, E, _inv_body, 0)

        def _init_body(i, _):
            par_s[i] = i
            deg_s[i] = 0
            vis_s[i] = 0
            adj_s[4 * i] = 0
            adj_s[4 * i + 1] = 0
            adj_s[4 * i + 2] = 0
            adj_s[4 * i + 3] = 0
            return 0

        lax.fori_loop(0, P, _init_body, 0)

        def _find(x0):
            def cond(carry):
                xx, pp = carry
                return pp != xx

            def body(carry):
                xx, pp = carry
                g = par_s[pp]
                par_s[xx] = g
                return g, par_s[g]

            root, _ = lax.while_loop(cond, body, (x0, par_s[x0]))
            return root

        def _kruskal_body(e, _):
            u = su_s[e]
            v = dv_s[e]
            ru = _find(u)
            rv = _find(v)

            @pl.when(ru != rv)
            def _():
                par_s[ru] = rv
                du = deg_s[u]
                adj_s[u * 4 + du] = v
                deg_s[u] = du + 1
                dvn = deg_s[v]
                adj_s[v * 4 + dvn] = u
                deg_s[v] = dvn + 1

            return 0

        lax.fori_loop(0, E, _kruskal_body, 0)

        semrow = sem_ref[pl.ds(b, 1), :]
        mx = jnp.max(semrow)
        root = jnp.min(jnp.where(semrow == mx, iota_p, 100000)).astype(jnp.int32)

        ord_s[0] = root
        pb_s[0] = jnp.int32(-1)
        vis_s[root] = jnp.int32(1)

        def _bfs_body(i, tail):
            v = ord_s[i]
            dvn = deg_s[v]
            for k in range(4):
                u = adj_s[v * 4 + k]
                take = (k < dvn) & (vis_s[u] == 0)

                @pl.when(take)
                def _():
                    vis_s[u] = jnp.int32(1)
                    ord_s[tail] = u
                    pb_s[tail] = i

                tail = jnp.where(take, tail + 1, tail)
            return tail

        lax.fori_loop(0, P, _bfs_body, jnp.int32(1))

        def _scan_body(i, _):
            v = ord_s[i]
            pbi = pb_s[i]
            row = boff + v
            dvec = delta_ref[pl.ds(row, 1), :]
            xvec = x_ref[pl.ds(row, 1), :]
            b_col = bt_ref[pl.ds(row * D_STATE, D_STATE), :]
            c_col = ct_ref[pl.ds(row * D_STATE, D_STATE), :]
            dx = dvec * xvec
            rooted = pbi >= 0
            hpb = jnp.maximum(pbi, 0) * D_STATE
            ab = jnp.exp(dvec * aneg_blk)
            hp = jnp.where(rooted, hs_v[pl.ds(hpb, D_STATE), :],
                           jnp.float32(0.0))
            hblk = ab * hp + dx * b_col
            hs_v[pl.ds(i * D_STATE, D_STATE), :] = hblk
            y = dp_row * xvec + jnp.sum(hblk * c_col, axis=0, keepdims=True)
            y_ref[pl.ds(row, 1), :] = y
            return 0

        lax.fori_loop(0, P, _scan_body, 0)


def _run_tc_tree(w_edges, sem2, delta, X, bt, ct, at2, dp_row):
    src_c = jnp.pad(jnp.asarray(_SRC_E), (0, EPAD - E))
    dst_c = jnp.pad(jnp.asarray(_DST_E), (0, EPAD - E))
    smem = pl.BlockSpec(memory_space=pltpu.SMEM)
    vm = pl.BlockSpec()
    return pl.pallas_call(
        _tc_tree,
        out_shape=jax.ShapeDtypeStruct((ROWS, D_F), jnp.float32),
        in_specs=[vm, smem, smem, smem, vm, vm, vm, vm, vm, vm, vm],
        scratch_shapes=[
            pltpu.SMEM((EPAD,), jnp.int32),
            pltpu.SMEM((EPAD,), jnp.int32),
            pltpu.SMEM((PPAD,), jnp.int32),
            pltpu.SMEM((PPAD,), jnp.int32),
            pltpu.SMEM((4 * PPAD,), jnp.int32),
            pltpu.SMEM((PPAD,), jnp.int32),
            pltpu.SMEM((PPAD,), jnp.int32),
            pltpu.SMEM((PPAD,), jnp.int32),
            pltpu.VMEM((P * D_STATE, D_F), jnp.float32),
        ],
    )(w_edges, w_edges, src_c, dst_c, sem2, delta, X, bt, ct, at2, dp_row)


def kernel(images, g_task, W1, b1, W2, b2, ln1_g, ln1_b, Wgate, Wgp, A_log,
           Dp, WB, bB, WC, bC, Wd, bd, Wo, bo, lno_g, lno_b):
    B, C, H, W = images.shape
    nH, nW = H // PS, W // PS
    x = images.reshape(B, C, nH, PS, nW, PS).transpose(0, 2, 4, 1, 3, 5)
    x = x.reshape(B * nH * nW, C * PS * PS)

    Wbc = jnp.concatenate([WB, WC], axis=1)
    bbc = jnp.concatenate([bB, bC]).reshape(1, 32)

    fr = pl.pallas_call(
        _tc_front,
        out_shape=[
            jax.ShapeDtypeStruct((ROWS, D_F), jnp.float32),
            jax.ShapeDtypeStruct((ROWS, D_F), jnp.float32),
            jax.ShapeDtypeStruct((ROWS, 32), jnp.float32),
            jax.ShapeDtypeStruct((ROWS, 1), jnp.float32),
            jax.ShapeDtypeStruct((ROWS, 1), jnp.float32),
            jax.ShapeDtypeStruct((ROWS, 1), jnp.float32),
        ],
    )(x, g_task, W1, b1.reshape(1, -1), W2, b2.reshape(1, -1),
      ln1_g.reshape(1, -1), ln1_b.reshape(1, -1), Wgate, Wgp, Wd,
      bd.reshape(1, -1), Wbc, bbc)
    X, delta, bc, sem, wr, wd = fr

    base = jnp.asarray(np.arange(NB)[:, None] * P + _SRC_E[None, :])
    wre = wr[:, 0][base]
    wde = wd[:, 0][base]
    w_edges = jnp.where(jnp.asarray(_IS_R)[None, :], wre, wde)
    w_edges = jnp.pad(w_edges, ((0, 0), (0, EPAD - E)),
                      constant_values=-1e30)
    sem2 = jnp.pad(sem[:, 0].reshape(NB, P), ((0, 0), (0, 256 - P)),
                   constant_values=-1e30)

    bt = bc[:, :D_STATE].reshape(ROWS * D_STATE, 1)
    ct = bc[:, D_STATE:].reshape(ROWS * D_STATE, 1)
    Y = _run_tc_tree(w_edges, sem2, delta, X, bt, ct, A_log.T,
                     Dp.reshape(1, D_F))

    Yr = Y
    out = pl.pallas_call(
        _tc_out,
        out_shape=jax.ShapeDtypeStruct((ROWS, D_F), jnp.float32),
    )(Yr, Wo, bo.reshape(1, -1),
      lno_g.reshape(1, -1), lno_b.reshape(1, -1))
    return out.reshape(NB, P, D_F)


# speedup vs baseline: 1890.7378x; 1.5432x over previous
import functools
import numpy as np
import jax
import jax.numpy as jnp
from jax import lax
from jax.experimental import pallas as pl
from jax.experimental.pallas import tpu as pltpu

PS = 16
D_F = 256
D_STATE = 16
ALPHA = 0.5
GH = 14
GW = 14
P = GH * GW
NB = 2
ROWS = NB * P
E = 2 * GH * GW - GH - GW
EPAD = 384
PPAD = 224
NCHUNKS_E = EPAD // 16
NCHUNKS_P = PPAD // 16


def _grid_edge_list():
    s, d = [], []
    for i in range(GH):
        for j in range(GW):
            u = i * GW + j
            if j + 1 < GW:
                s.append(u)
                d.append(u + 1)
            if i + 1 < GH:
                s.append(u)
                d.append(u + GW)
    return np.array(s, np.int32), np.array(d, np.int32)


_SRC_E, _DST_E = _grid_edge_list()
_IS_R = (_DST_E == _SRC_E + 1)


def _tc_front(x_ref, g_ref, W1_ref, b1_ref, W2_ref, b2_ref, g1_ref, bl1_ref,
              Wgate_ref, Wgp_ref, Wd_ref, bd_ref, Wbc_ref, bbc_ref,
              x_out, delta_out, bc_out, sem_out, wr_out, wd_out):
    x = x_ref[...]
    h = jnp.dot(x, W1_ref[...], preferred_element_type=jnp.float32) + b1_ref[...]
    h = 0.5 * h * (1.0 + lax.erf(h * np.float32(1.0 / np.sqrt(2.0))))
    f = jnp.dot(h, W2_ref[...], preferred_element_type=jnp.float32) + b2_ref[...]
    m = f.mean(-1, keepdims=True)
    v = ((f - m) ** 2).mean(-1, keepdims=True)
    f = (f - m) * lax.rsqrt(v + np.float32(1e-5)) * g1_ref[...] + bl1_ref[...]

    g = g_ref[...]
    gg = jnp.dot(g, Wgate_ref[...], preferred_element_type=jnp.float32)
    gp = jnp.dot(g, Wgp_ref[...], preferred_element_type=jnp.float32)
    gn = gg / jnp.maximum(jnp.sqrt((gg * gg).sum(-1, keepdims=True)), np.float32(1e-12))
    fn = f / jnp.maximum(jnp.sqrt((f * f).sum(-1, keepdims=True)), np.float32(1e-12))

    ridx = lax.broadcasted_iota(jnp.int32, (ROWS, D_F), 0)
    img0 = ridx < P
    gnsel = jnp.where(img0, gn[0:1, :], gn[1:2, :])
    gpsel = jnp.where(img0, gp[0:1, :], gp[1:2, :])
    sem = (fn * gnsel).sum(-1, keepdims=True)
    X = f + sem * gpsel

    xw = jnp.dot(X, Wd_ref[...], preferred_element_type=jnp.float32) + bd_ref[...]
    delta = jnp.maximum(xw, 0.0) + jnp.log1p(jnp.exp(-jnp.abs(xw)))
    bc = jnp.dot(X, Wbc_ref[...], preferred_element_type=jnp.float32) + bbc_ref[...]

    wr = (fn[:-1] * fn[1:]).sum(-1, keepdims=True) + \
        np.float32(ALPHA) * (sem[:-1] * sem[1:])
    wd = (fn[:-GW] * fn[GW:]).sum(-1, keepdims=True) + \
        np.float32(ALPHA) * (sem[:-GW] * sem[GW:])

    x_out[...] = X
    delta_out[...] = delta
    bc_out[...] = bc
    sem_out[...] = sem
    wr_out[...] = jnp.concatenate([wr, jnp.zeros((1, 1), jnp.float32)], axis=0)
    wd_out[...] = jnp.concatenate([wd, jnp.zeros((GW, 1), jnp.float32)], axis=0)


def _tc_out(y_ref, Wo_ref, bo_ref, g_ref, b_ref, o_ref):
    o = jnp.dot(y_ref[...], Wo_ref[...], preferred_element_type=jnp.float32) + bo_ref[...]
    m = o.mean(-1, keepdims=True)
    v = ((o - m) ** 2).mean(-1, keepdims=True)
    o_ref[...] = (o - m) * lax.rsqrt(v + np.float32(1e-5)) * g_ref[...] + b_ref[...]


def _tc_tree(w_v_ref, w_c_ref, src_ref, dst_ref, sem_ref, delta_ref, x_ref,
             bt_ref, ct_ref, at_ref, dp_ref, y_ref,
             su_s, dv_s, par_s, deg_s, adj_s, vis_s, ord_s, pb_s, hs_v,
             pos_v, pos_s, dma_sem):
    iota_e = lax.broadcasted_iota(jnp.int32, (1, EPAD), 1)
    iota_p = lax.broadcasted_iota(jnp.int32, (1, 256), 1)
    dp_row = dp_ref[...]
    aneg_blk = -jnp.exp(at_ref[...])

    iota_col = lax.broadcasted_iota(jnp.int32, (EPAD, EPAD), 0)
    iota_row = lax.broadcasted_iota(jnp.int32, (EPAD, EPAD), 1)

    for b in range(NB):
        boff = b * P
        wrow = w_v_ref[pl.ds(b, 1), :]
        wcol = w_c_ref[pl.ds(b * EPAD, EPAD), :]

        prec = (wrow > wcol) | ((wrow == wcol) & (iota_row < iota_col))
        pos = jnp.sum(jnp.where(prec, 1, 0).astype(jnp.int32), axis=1,
                      keepdims=True)
        pos_v[...] = pos
        copy = pltpu.make_async_copy(pos_v, pos_s, dma_sem)
        copy.start()
        copy.wait()

        def _sort_body(e, _):
            p = pos_s[e, 0]
            su_s[p] = src_ref[e]
            dv_s[p] = dst_ref[e]
            return 0

        lax.fori_loop(0, E, _sort_body, 0)

        def _init_body(i, _):
            par_s[i] = i
            deg_s[i] = 0
            vis_s[i] = 0
            adj_s[4 * i] = 0
            adj_s[4 * i + 1] = 0
            adj_s[4 * i + 2] = 0
            adj_s[4 * i + 3] = 0
            return 0

        lax.fori_loop(0, P, _init_body, 0)

        def _find(x0):
            def cond(carry):
                xx, pp = carry
                return pp != xx

            def body(carry):
                xx, pp = carry
                g = par_s[pp]
                par_s[xx] = g
                return g, par_s[g]

            root, _ = lax.while_loop(cond, body, (x0, par_s[x0]))
            return root

        def _kruskal_body(e, _):
            u = su_s[e]
            v = dv_s[e]
            ru = _find(u)
            rv = _find(v)

            @pl.when(ru != rv)
            def _():
                par_s[ru] = rv
                du = deg_s[u]
                adj_s[u * 4 + du] = v
                deg_s[u] = du + 1
                dvn = deg_s[v]
                adj_s[v * 4 + dvn] = u
                deg_s[v] = dvn + 1

            return 0

        lax.fori_loop(0, E, _kruskal_body, 0)

        semrow = sem_ref[pl.ds(b, 1), :]
        mx = jnp.max(semrow)
        root = jnp.min(jnp.where(semrow == mx, iota_p, 100000)).astype(jnp.int32)

        ord_s[0] = root
        pb_s[0] = jnp.int32(-1)
        vis_s[root] = jnp.int32(1)

        def _bfs_body(i, tail):
            v = ord_s[i]
            dvn = deg_s[v]
            for k in range(4):
                u = adj_s[v * 4 + k]
                take = (k < dvn) & (vis_s[u] == 0)

                @pl.when(take)
                def _():
                    vis_s[u] = jnp.int32(1)
                    ord_s[tail] = u
                    pb_s[tail] = i

                tail = jnp.where(take, tail + 1, tail)
            return tail

        lax.fori_loop(0, P, _bfs_body, jnp.int32(1))

        def _scan_body(i, _):
            v = ord_s[i]
            pbi = pb_s[i]
            row = boff + v
            dvec = delta_ref[pl.ds(row, 1), :]
            xvec = x_ref[pl.ds(row, 1), :]
            b_col = bt_ref[pl.ds(row * D_STATE, D_STATE), :]
            c_col = ct_ref[pl.ds(row * D_STATE, D_STATE), :]
            dx = dvec * xvec
            rooted = pbi >= 0
            hpb = jnp.maximum(pbi, 0) * D_STATE
            ab = jnp.exp(dvec * aneg_blk)
            hp = jnp.where(rooted, hs_v[pl.ds(hpb, D_STATE), :],
                           jnp.float32(0.0))
            hblk = ab * hp + dx * b_col
            hs_v[pl.ds(i * D_STATE, D_STATE), :] = hblk
            y = dp_row * xvec + jnp.sum(hblk * c_col, axis=0, keepdims=True)
            y_ref[pl.ds(row, 1), :] = y
            return 0

        lax.fori_loop(0, P, _scan_body, 0)


def _run_tc_tree(w_edges, sem2, delta, X, bt, ct, at2, dp_row):
    src_c = jnp.pad(jnp.asarray(_SRC_E), (0, EPAD - E))
    dst_c = jnp.pad(jnp.asarray(_DST_E), (0, EPAD - E))
    smem = pl.BlockSpec(memory_space=pltpu.SMEM)
    vm = pl.BlockSpec()
    return pl.pallas_call(
        _tc_tree,
        out_shape=jax.ShapeDtypeStruct((ROWS, D_F), jnp.float32),
        in_specs=[vm, vm, smem, smem, vm, vm, vm, vm, vm, vm, vm],
        scratch_shapes=[
            pltpu.SMEM((EPAD,), jnp.int32),
            pltpu.SMEM((EPAD,), jnp.int32),
            pltpu.SMEM((PPAD,), jnp.int32),
            pltpu.SMEM((PPAD,), jnp.int32),
            pltpu.SMEM((4 * PPAD,), jnp.int32),
            pltpu.SMEM((PPAD,), jnp.int32),
            pltpu.SMEM((PPAD,), jnp.int32),
            pltpu.SMEM((PPAD,), jnp.int32),
            pltpu.VMEM((P * D_STATE, D_F), jnp.float32),
            pltpu.VMEM((EPAD, 1), jnp.int32),
            pltpu.SMEM((EPAD, 1), jnp.int32),
            pltpu.SemaphoreType.DMA,
        ],
    )(w_edges, w_edges.reshape(NB * EPAD, 1), src_c, dst_c, sem2, delta, X,
      bt, ct, at2, dp_row)


def kernel(images, g_task, W1, b1, W2, b2, ln1_g, ln1_b, Wgate, Wgp, A_log,
           Dp, WB, bB, WC, bC, Wd, bd, Wo, bo, lno_g, lno_b):
    B, C, H, W = images.shape
    nH, nW = H // PS, W // PS
    x = images.reshape(B, C, nH, PS, nW, PS).transpose(0, 2, 4, 1, 3, 5)
    x = x.reshape(B * nH * nW, C * PS * PS)

    Wbc = jnp.concatenate([WB, WC], axis=1)
    bbc = jnp.concatenate([bB, bC]).reshape(1, 32)

    fr = pl.pallas_call(
        _tc_front,
        out_shape=[
            jax.ShapeDtypeStruct((ROWS, D_F), jnp.float32),
            jax.ShapeDtypeStruct((ROWS, D_F), jnp.float32),
            jax.ShapeDtypeStruct((ROWS, 32), jnp.float32),
            jax.ShapeDtypeStruct((ROWS, 1), jnp.float32),
            jax.ShapeDtypeStruct((ROWS, 1), jnp.float32),
            jax.ShapeDtypeStruct((ROWS, 1), jnp.float32),
        ],
    )(x, g_task, W1, b1.reshape(1, -1), W2, b2.reshape(1, -1),
      ln1_g.reshape(1, -1), ln1_b.reshape(1, -1), Wgate, Wgp, Wd,
      bd.reshape(1, -1), Wbc, bbc)
    X, delta, bc, sem, wr, wd = fr

    base = jnp.asarray(np.arange(NB)[:, None] * P + _SRC_E[None, :])
    wre = wr[:, 0][base]
    wde = wd[:, 0][base]
    w_edges = jnp.where(jnp.asarray(_IS_R)[None, :], wre, wde)
    w_edges = jnp.pad(w_edges, ((0, 0), (0, EPAD - E)),
                      constant_values=-1e30)
    sem2 = jnp.pad(sem[:, 0].reshape(NB, P), ((0, 0), (0, 256 - P)),
                   constant_values=-1e30)

    bt = bc[:, :D_STATE].reshape(ROWS * D_STATE, 1)
    ct = bc[:, D_STATE:].reshape(ROWS * D_STATE, 1)
    Y = _run_tc_tree(w_edges, sem2, delta, X, bt, ct, A_log.T,
                     Dp.reshape(1, D_F))

    Yr = Y
    out = pl.pallas_call(
        _tc_out,
        out_shape=jax.ShapeDtypeStruct((ROWS, D_F), jnp.float32),
    )(Yr, Wo, bo.reshape(1, -1),
      lno_g.reshape(1, -1), lno_b.reshape(1, -1))
    return out.reshape(NB, P, D_F)
